# baseline (speedup 1.0000x reference)
import jax
import jax.numpy as jnp
from jax import lax
from jax.experimental import pallas as pl
from jax.experimental.pallas import tpu as pltpu
from jax.experimental.pallas import tpu_sc as plsc

_VOCAB = 100000
_EMBED_DIM = 64
_BATCH = 4096
_HIST = 50

_NW = 32
_BPW = _BATCH // _NW
_L = 16


def _transpose_chunk(rows_v, tr_v):
    for b in range(_BPW):
        bvec = jnp.full((_L,), b, jnp.int32)
        for q in range(_EMBED_DIM // _L):
            vals = rows_v[b, pl.ds(_L * q, _L)]
            dvec = lax.iota(jnp.int32, _L) + _L * q
            plsc.store_scatter(tr_v, [dvec, bvec], vals)


def _body(idxt_hbm, table_hbm, out_hbm, idx_v, rows_a, rows_b, tr_a, tr_b,
          gsem_a, gsem_b, wsem_a, wsem_b):
    wid = lax.axis_index("s") * 2 + lax.axis_index("c")
    b0 = wid * _BPW
    pltpu.sync_copy(idxt_hbm.at[:, pl.ds(b0, _BPW)], idx_v)

    rows = (rows_a, rows_b)
    trs = (tr_a, tr_b)
    gsems = (gsem_a, gsem_b)
    wsems = (wsem_a, wsem_b)

    def gather(h, slot):
        return pltpu.make_async_copy(
            table_hbm.at[idx_v.at[h]], rows[slot], gsems[slot])

    def write(h, slot):
        return pltpu.make_async_copy(
            trs[slot],
            out_hbm.at[pl.ds(h * _EMBED_DIM, _EMBED_DIM), pl.ds(b0, _BPW)],
            wsems[slot])

    gather(0, 0).start()
    gather(1, 1).start()

    def step(g, last):
        for slot in range(2):
            h = 2 * g + slot
            gather(h, slot).wait()

            @pl.when(g > 0)
            def _():
                write(h, slot).wait()

            _transpose_chunk(rows[slot], trs[slot])
            if not last:
                gather(h + 2, slot).start()
            write(h, slot).start()

    @pl.loop(0, _HIST // 2 - 1)
    def grp(g):
        step(g, False)

    step(_HIST // 2 - 1, True)
    write(0, 0).wait()
    write(1, 1).wait()


@jax.jit
def _embed(idxt, table):
    mesh = plsc.VectorSubcoreMesh(core_axis_name="c", subcore_axis_name="s")
    f = pl.kernel(
        _body,
        out_type=jax.ShapeDtypeStruct((_HIST * _EMBED_DIM, _BATCH),
                                      jnp.float32),
        mesh=mesh,
        scratch_types=[
            pltpu.VMEM((_HIST, _BPW), jnp.int32),
            pltpu.VMEM((_BPW, _EMBED_DIM), jnp.float32),
            pltpu.VMEM((_BPW, _EMBED_DIM), jnp.float32),
            pltpu.VMEM((_EMBED_DIM, _BPW), jnp.float32),
            pltpu.VMEM((_EMBED_DIM, _BPW), jnp.float32),
        ] + [pltpu.SemaphoreType.DMA] * 4,
        compiler_params=pltpu.CompilerParams(
            use_tc_tiling_on_sc=False, needs_layout_passes=False),
    )
    return f(idxt, table)


def kernel(input, table):
    idxt = input.T.astype(jnp.int32)
    out = _embed(idxt, table)
    return out.reshape(_HIST, _EMBED_DIM, _BATCH).transpose(2, 0, 1)
